# Initial kernel scaffold; baseline (speedup 1.0000x reference)
import jax
import jax.numpy as jnp
from jax.experimental import pallas as pl


def kernel(inputs, table, W1, b1):
    raise NotImplementedError("write your pallas kernel here")


# speedup vs baseline: 2.5870x; 2.5870x over previous
import functools

import jax
import jax.numpy as jnp
from jax import lax
from jax.experimental import pallas as pl
from jax.experimental.pallas import tpu as pltpu
from jax.experimental.pallas import tpu_sc as plsc

VOCAB = 1000000
EMB = 64
BATCH = 16384
HIST = 50
NUM_CLASSES = 1000

NC = 2
NS = 16
NW = NC * NS
BPW = BATCH // NW
SUB = 128
NSUB = BPW // SUB
NSTREAM = NSUB * HIST


def _pool_body(idx_hbm, table_hbm, agg_hbm, idx_v, acc_v, sem):
  c = lax.axis_index("c")
  s = lax.axis_index("s")
  wid = s * NC + c
  base = wid * BPW

  pltpu.sync_copy(idx_hbm.at[wid], idx_v)

  zero = jnp.zeros((16,), jnp.float32)

  def zbody(i, carry):
    acc_v[i // (EMB // 16), pl.ds((i % (EMB // 16)) * 16, 16)] = zero
    return carry

  lax.fori_loop(0, BPW * (EMB // 16), zbody, 0)

  def gbody(r, carry):
    sub = r // HIST
    dst = acc_v.at[pl.ds(sub * SUB, SUB), :]
    pltpu.async_copy(table_hbm.at[idx_v.at[r]], dst, sem, add=True)
    return carry

  lax.fori_loop(0, NSTREAM, gbody, 0)

  def wbody(r, carry):
    pltpu.make_async_copy(
        table_hbm.at[idx_v.at[0]], acc_v.at[pl.ds(0, SUB), :], sem
    ).wait()
    return carry

  lax.fori_loop(0, NSTREAM, wbody, 0)

  pltpu.sync_copy(acc_v, agg_hbm.at[pl.ds(base, BPW), :])


_pool = functools.partial(
    pl.kernel,
    out_type=jax.ShapeDtypeStruct((BATCH, EMB), jnp.float32),
    mesh=plsc.VectorSubcoreMesh(core_axis_name="c", subcore_axis_name="s"),
    scratch_types=[
        pltpu.VMEM((NSTREAM, SUB), jnp.int32),
        pltpu.VMEM((BPW, EMB), jnp.float32),
        pltpu.SemaphoreType.DMA,
    ],
    compiler_params=pltpu.CompilerParams(use_tc_tiling_on_sc=False),
)(_pool_body)


def _mm_body(a_ref, w_ref, b_ref, o_ref):
  o_ref[...] = (
      lax.dot_general(
          a_ref[...],
          w_ref[...],
          (((1,), (1,)), ((), ())),
          preferred_element_type=jnp.float32,
      )
      + b_ref[...]
  )


_BM = 1024

_mm = pl.pallas_call(
    _mm_body,
    grid=(BATCH // _BM,),
    in_specs=[
        pl.BlockSpec((_BM, EMB), lambda i: (i, 0)),
        pl.BlockSpec((NUM_CLASSES, EMB), lambda i: (0, 0)),
        pl.BlockSpec((1, NUM_CLASSES), lambda i: (0, 0)),
    ],
    out_specs=pl.BlockSpec((_BM, NUM_CLASSES), lambda i: (i, 0)),
    out_shape=jax.ShapeDtypeStruct((BATCH, NUM_CLASSES), jnp.float32),
)


@jax.jit
def kernel(inputs, table, W1, b1):
  idx = inputs.astype(jnp.int32)
  idx3 = (
      idx.reshape(NW, NSUB, SUB, HIST)
      .transpose(0, 1, 3, 2)
      .reshape(NW, NSTREAM, SUB)
  )
  agg = _pool(idx3, table)
  return _mm(agg, W1, b1.reshape(1, NUM_CLASSES))
